# baseline (speedup 1.0000x reference)
import functools

import jax
import jax.numpy as jnp
from jax import lax
from jax.experimental import pallas as pl
from jax.experimental.pallas import tpu as pltpu
from jax.experimental.pallas import tpu_sc as plsc


@functools.lru_cache(maxsize=None)
def _build_gather(V, D, B):
    info = plsc.get_sparse_core_info()
    NC, NS = info.num_cores, info.num_subcores
    NW = NC * NS
    assert B % NW == 0
    b_per_w = B // NW
    CHUNK = 1024
    assert b_per_w % CHUNK == 0
    n_chunks = b_per_w // CHUNK
    mesh = plsc.VectorSubcoreMesh(core_axis_name="c", subcore_axis_name="s")

    @functools.partial(
        pl.kernel,
        mesh=mesh,
        out_type=jax.ShapeDtypeStruct((B, D), jnp.float32),
        scratch_types=[
            pltpu.VMEM((CHUNK,), jnp.int32),
            pltpu.VMEM((CHUNK, D), jnp.float32),
            pltpu.SemaphoreType.DMA,
        ],
        compiler_params=pltpu.CompilerParams(use_tc_tiling_on_sc=False),
    )
    def gather_k(table_hbm, idx_hbm, out_hbm, idx_v, rows_v, sem):
        wid = lax.axis_index("s") * NC + lax.axis_index("c")
        base = wid * b_per_w

        def step(i, carry):
            off = base + i * CHUNK
            pltpu.sync_copy(idx_hbm.at[pl.ds(off, CHUNK)], idx_v)
            pltpu.async_copy(table_hbm.at[idx_v], rows_v, sem).wait()
            pltpu.sync_copy(rows_v, out_hbm.at[pl.ds(off, CHUNK)])
            return carry

        lax.fori_loop(0, n_chunks, step, 0)

    return gather_k


def kernel(input, weight):
    B0, B1 = input.shape
    V, D = weight.shape
    idx = input.reshape(-1).astype(jnp.int32)
    out = _build_gather(V, D, idx.shape[0])(weight, idx)
    return out.reshape(B0, B1, D)


# speedup vs baseline: 1.0173x; 1.0173x over previous
import functools

import jax
import jax.numpy as jnp
from jax import lax
from jax.experimental import pallas as pl
from jax.experimental.pallas import tpu as pltpu
from jax.experimental.pallas import tpu_sc as plsc

N_BUF = 4
K_LAG = 2
CHUNK = 400


@functools.lru_cache(maxsize=None)
def _build_gather(V, D, B):
    info = plsc.get_sparse_core_info()
    NC, NS = info.num_cores, info.num_subcores
    NW = NC * NS
    assert B % NW == 0
    b_per_w = B // NW
    assert b_per_w % CHUNK == 0
    n_chunks = b_per_w // CHUNK
    assert n_chunks % N_BUF == 0
    n_groups = n_chunks // N_BUF
    mesh = plsc.VectorSubcoreMesh(core_axis_name="c", subcore_axis_name="s")

    @functools.partial(
        pl.kernel,
        mesh=mesh,
        out_type=jax.ShapeDtypeStruct((B, D), jnp.float32),
        scratch_types=[
            pltpu.VMEM((b_per_w,), jnp.int32),
            [pltpu.VMEM((CHUNK, D), jnp.float32) for _ in range(N_BUF)],
            [pltpu.SemaphoreType.DMA for _ in range(N_BUF)],
            [pltpu.SemaphoreType.DMA for _ in range(N_BUF)],
        ],
        compiler_params=pltpu.CompilerParams(use_tc_tiling_on_sc=False),
    )
    def gather_k(table_hbm, idx_hbm, out_hbm, idx_all, rows, gsems, ssems):
        wid = lax.axis_index("s") * NC + lax.axis_index("c")
        base = wid * b_per_w
        pltpu.sync_copy(idx_hbm.at[pl.ds(base, b_per_w)], idx_all)

        def fire_gather(i, b):
            pltpu.make_async_copy(
                table_hbm.at[idx_all.at[pl.ds(i * CHUNK, CHUNK)]],
                rows[b], gsems[b]).start()

        def wait_gather(b):
            pltpu.make_async_copy(
                table_hbm.at[idx_all.at[pl.ds(0, CHUNK)]],
                rows[b], gsems[b]).wait()

        def fire_store(i, b):
            pltpu.make_async_copy(
                rows[b], out_hbm.at[pl.ds(base + i * CHUNK, CHUNK)],
                ssems[b]).start()

        def wait_store(b):
            pltpu.make_async_copy(
                rows[b], out_hbm.at[pl.ds(base, CHUNK)], ssems[b]).wait()

        for b in range(N_BUF):
            fire_gather(b, b)

        def group(g, carry):
            for b in range(N_BUF):
                i = g * N_BUF + b
                wait_gather(b)
                fire_store(i, b)
                j = i - K_LAG
                bj = (b - K_LAG) % N_BUF

                @pl.when(jnp.logical_and(j >= 0, j + N_BUF < n_chunks))
                def _():
                    wait_store(bj)
                    fire_gather(j + N_BUF, bj)

            return carry

        lax.fori_loop(0, n_groups, group, 0)
        for b in range(N_BUF):
            wait_store(b)

    return gather_k


def kernel(input, weight):
    B0, B1 = input.shape
    V, D = weight.shape
    idx = input.reshape(-1).astype(jnp.int32)
    out = _build_gather(V, D, idx.shape[0])(weight, idx)
    return out.reshape(B0, B1, D)
